# baseline (speedup 1.0000x reference)
import functools

import jax
import jax.numpy as jnp
from jax import lax
from jax.experimental import pallas as pl
from jax.experimental.pallas import tpu as pltpu
from jax.experimental.pallas import tpu_sc as plsc

N = 500000
CHUNK = 2000
NUM_CHUNKS = N // CHUNK
L = 16
GROUPS = CHUNK // L
NWORKERS = 32
ITERS = -(-NUM_CHUNKS // NWORKERS)
DEPTH_WEIGHT = 0.1
EPS = 1e-6


def _cross(a, b):
    return (a[1] * b[2] - a[2] * b[1],
            a[2] * b[0] - a[0] * b[2],
            a[0] * b[1] - a[1] * b[0])


def _quat_rotate(q, v):
    qv = (q[0], q[1], q[2])
    qw = q[3]
    uv = _cross(qv, v)
    uuv = _cross(qv, uv)
    return tuple(v[i] + 2.0 * (qw * uv[i] + uuv[i]) for i in range(3))


def _quat_mul(q1, q2):
    x1, y1, z1, w1 = q1
    x2, y2, z2, w2 = q2
    return (w1 * x2 + x1 * w2 + y1 * z2 - z1 * y2,
            w1 * y2 - x1 * z2 + y1 * w2 + z1 * x2,
            w1 * z2 + x1 * y2 - y1 * x2 + z1 * w2,
            w1 * w2 - x1 * x2 - y1 * y2 - z1 * z2)


NUM_PTS = 100000
PT_SEGS = NUM_PTS // CHUNK


def _sc_body(p2x_hbm, p2y_hbm, gidx_hbm, midx_hbm, cidx_hbm, pidx_hbm,
             depth_hbm, cam_hbm, rel_hbm, p3x_hbm, p3y_hbm, p3z_hbm,
             ref_hbm, out_hbm, tbl_hbm,
             gidx_v0, midx_v0, cidx_v0, pidx_v0, p2x_v0, p2y_v0, depth_v0,
             pts_v0, ref_v0, out0_v0, out1_v0, out2_v0,
             gidx_v1, midx_v1, cidx_v1, pidx_v1, p2x_v1, p2y_v1, depth_v1,
             pts_v1, ref_v1, out0_v1, out1_v1, out2_v1,
             cam_v, rel_v, sem_lin0, sem_gat0, sem_lin1, sem_gat1):
    cid = lax.axis_index("c")
    sid = lax.axis_index("s")
    wid = sid * 2 + cid
    pts_hbm = tbl_hbm.at[cid]
    pltpu.sync_copy(cam_hbm, cam_v)
    pltpu.sync_copy(rel_hbm, rel_v)

    for it2 in range(-(-PT_SEGS // 16)):
        seg = sid + it2 * 16

        @pl.when(seg < PT_SEGS)
        def _build():
            base = seg * CHUNK
            pltpu.sync_copy(p3x_hbm.at[pl.ds(base, CHUNK)], p2x_v0)
            pltpu.sync_copy(p3y_hbm.at[pl.ds(base, CHUNK)], p2y_v0)
            pltpu.sync_copy(p3z_hbm.at[pl.ds(base, CHUNK)], depth_v0)

            @plsc.parallel_loop(0, GROUPS, unroll=2)
            def _rows(j):
                rows = j * L + lax.iota(jnp.int32, L)
                for col, src in ((0, p2x_v0), (1, p2y_v0), (2, depth_v0)):
                    cols = jnp.full((L,), col, jnp.int32)
                    plsc.store_scatter(pts_v0, [rows, cols],
                                       src[pl.ds(j * L, L)])

            pltpu.sync_copy(pts_v0, pts_hbm.at[pl.ds(base, CHUNK)])

    plsc.subcore_barrier()

    sets = [
        dict(gidx=gidx_v0, midx=midx_v0, cidx=cidx_v0, pidx=pidx_v0,
             p2x=p2x_v0, p2y=p2y_v0, depth=depth_v0, pts=pts_v0, ref=ref_v0,
             out0=out0_v0, out1=out1_v0, out2=out2_v0,
             sem_lin=sem_lin0, sem_gat=sem_gat0),
        dict(gidx=gidx_v1, midx=midx_v1, cidx=cidx_v1, pidx=pidx_v1,
             p2x=p2x_v1, p2y=p2y_v1, depth=depth_v1, pts=pts_v1, ref=ref_v1,
             out0=out0_v1, out1=out1_v1, out2=out2_v1,
             sem_lin=sem_lin1, sem_gat=sem_gat1),
    ]

    def lin_pairs(c, s):
        base = c * CHUNK
        return [
            (gidx_hbm.at[pl.ds(base, CHUNK)], s["gidx"]),
            (midx_hbm.at[pl.ds(base, CHUNK)], s["midx"]),
            (cidx_hbm.at[pl.ds(base, CHUNK)], s["cidx"]),
            (pidx_hbm.at[pl.ds(base, CHUNK)], s["pidx"]),
            (p2x_hbm.at[pl.ds(base, CHUNK)], s["p2x"]),
            (p2y_hbm.at[pl.ds(base, CHUNK)], s["p2y"]),
            (depth_hbm.at[pl.ds(base, CHUNK)], s["depth"]),
        ]

    def start_stage1(c, s):
        for src, dst in lin_pairs(c, s):
            pltpu.async_copy(src, dst, s["sem_lin"])

    def wait_stage1(c, s):
        for src, dst in lin_pairs(c, s):
            pltpu.make_async_copy(src, dst, s["sem_lin"]).wait()

    def start_gathers(s):
        pltpu.async_copy(pts_hbm.at[s["pidx"]], s["pts"], s["sem_gat"])
        pltpu.async_copy(ref_hbm.at[s["gidx"]], s["ref"], s["sem_gat"])

    def wait_gathers(s):
        pltpu.make_async_copy(pts_hbm.at[s["pidx"]], s["pts"], s["sem_gat"]).wait()
        pltpu.make_async_copy(ref_hbm.at[s["gidx"]], s["ref"], s["sem_gat"]).wait()

    def compute(c, s):
        base = c * CHUNK

        @plsc.parallel_loop(0, GROUPS, unroll=4)
        def body(j):
            rows = j * L + lax.iota(jnp.int32, L)

            def lg(ref, idxv, col):
                cols = jnp.full((L,), col, jnp.int32)
                return plsc.load_gather(ref, [idxv, cols])

            rt = tuple(lg(s["ref"], rows, k) for k in range(3))
            rq = tuple(lg(s["ref"], rows, k) for k in range(3, 7))
            mi = s["midx"][pl.ds(j * L, L)]
            ci = s["cidx"][pl.ds(j * L, L)]
            lt = tuple(lg(rel_v, mi, k) for k in range(3))
            lq = tuple(lg(rel_v, mi, k) for k in range(3, 7))
            fx = lg(cam_v, ci, 0)
            fy = lg(cam_v, ci, 1)
            cx = lg(cam_v, ci, 2)
            cy = lg(cam_v, ci, 3)
            p = tuple(lg(s["pts"], rows, k) for k in range(3))

            rrt = _quat_rotate(lq, rt)
            t = tuple(lt[i] + rrt[i] for i in range(3))
            q = _quat_mul(lq, rq)
            pc = _quat_rotate(q, p)
            pcx = pc[0] + t[0]
            pcy = pc[1] + t[1]
            z = pc[2] + t[2]
            u = pcx / z * fx + cx
            v = pcy / z * fy + cy
            p2x = s["p2x"][pl.ds(j * L, L)]
            p2y = s["p2y"][pl.ds(j * L, L)]
            dref = s["depth"][pl.ds(j * L, L)]
            out0 = u - p2x
            out1 = v - p2y
            out2 = (1.0 / (z + EPS) - dref) * DEPTH_WEIGHT
            s["out0"][pl.ds(j * L, L)] = out0
            s["out1"][pl.ds(j * L, L)] = out1
            s["out2"][pl.ds(j * L, L)] = out2

        pltpu.sync_copy(s["out0"], out_hbm.at[0, pl.ds(base, CHUNK)])
        pltpu.sync_copy(s["out1"], out_hbm.at[1, pl.ds(base, CHUNK)])
        pltpu.sync_copy(s["out2"], out_hbm.at[2, pl.ds(base, CHUNK)])

    start_stage1(wid, sets[0])
    wait_stage1(wid, sets[0])
    start_gathers(sets[0])
    for it in range(ITERS):
        c = wid + it * NWORKERS
        cn = wid + (it + 1) * NWORKERS
        cur = sets[it % 2]
        nxt = sets[(it + 1) % 2]
        last = it == ITERS - 1

        def guarded(fn, cond):
            if cond is None:
                fn()
            else:
                pl.when(cond)(fn)

        if not last:
            next_cond = None if it + 1 < ITERS - 1 else (cn < NUM_CHUNKS)
            guarded(lambda: start_stage1(cn, nxt), next_cond)
            wait_gathers(cur)
            guarded(lambda: (wait_stage1(cn, nxt), start_gathers(nxt))[0],
                    next_cond)
            compute(c, cur)
        else:
            @pl.when(c < NUM_CHUNKS)
            def _tail():
                wait_gathers(cur)
                compute(c, cur)


_sc_call = functools.partial(
    pl.kernel,
    mesh=plsc.VectorSubcoreMesh(core_axis_name="c", subcore_axis_name="s"),
    out_type=(jax.ShapeDtypeStruct((3, N), jnp.float32),
              jax.ShapeDtypeStruct((2, NUM_PTS, 8), jnp.float32)),
    compiler_params=pltpu.CompilerParams(
        needs_layout_passes=False, use_tc_tiling_on_sc=False),
    scratch_types=(
        [
            pltpu.VMEM((CHUNK,), jnp.int32),
            pltpu.VMEM((CHUNK,), jnp.int32),
            pltpu.VMEM((CHUNK,), jnp.int32),
            pltpu.VMEM((CHUNK,), jnp.int32),
            pltpu.VMEM((CHUNK,), jnp.float32),
            pltpu.VMEM((CHUNK,), jnp.float32),
            pltpu.VMEM((CHUNK,), jnp.float32),
            pltpu.VMEM((CHUNK, 8), jnp.float32),
            pltpu.VMEM((CHUNK, 8), jnp.float32),
            pltpu.VMEM((CHUNK,), jnp.float32),
            pltpu.VMEM((CHUNK,), jnp.float32),
            pltpu.VMEM((CHUNK,), jnp.float32),
        ] * 2
        + [
            pltpu.VMEM((8, 4), jnp.float32),
            pltpu.VMEM((8, 8), jnp.float32),
            pltpu.SemaphoreType.DMA,
            pltpu.SemaphoreType.DMA,
            pltpu.SemaphoreType.DMA,
            pltpu.SemaphoreType.DMA,
        ]
    ),
)(_sc_body)


def kernel(points_2d, camera_indices, grouping_indices, point_indices,
           camera_pps, rel_poses, depths_ref, intrs, points_3d, ref_poses):
    gidx = grouping_indices[:, 0].astype(jnp.int32)
    midx = grouping_indices[:, 1].astype(jnp.int32)
    cidx = camera_indices.astype(jnp.int32)
    pidx = point_indices.astype(jnp.int32)
    p2x = points_2d[:, 0]
    p2y = points_2d[:, 1]
    p3x = points_3d[:, 0]
    p3y = points_3d[:, 1]
    p3z = points_3d[:, 2]
    ref8 = jnp.pad(ref_poses, ((0, 0), (0, 1)))
    rel8 = jnp.pad(rel_poses, ((0, 0), (0, 1)))
    cam4 = jnp.concatenate([intrs, camera_pps], axis=1)
    out, _ = _sc_call(p2x, p2y, gidx, midx, cidx, pidx, depths_ref,
                      cam4, rel8, p3x, p3y, p3z, ref8)
    return out.T


# speedup vs baseline: 1.2215x; 1.2215x over previous
import functools

import jax
import jax.numpy as jnp
from jax import lax
from jax.experimental import pallas as pl
from jax.experimental.pallas import tpu as pltpu
from jax.experimental.pallas import tpu_sc as plsc

N = 500000
CHUNK = 2000
NUM_CHUNKS = N // CHUNK
L = 16
GROUPS = CHUNK // L
NWORKERS = 32
ITERS = -(-NUM_CHUNKS // NWORKERS)
DEPTH_WEIGHT = 0.1
EPS = 1e-6


def _cross(a, b):
    return (a[1] * b[2] - a[2] * b[1],
            a[2] * b[0] - a[0] * b[2],
            a[0] * b[1] - a[1] * b[0])


def _quat_rotate(q, v):
    qv = (q[0], q[1], q[2])
    qw = q[3]
    uv = _cross(qv, v)
    uuv = _cross(qv, uv)
    return tuple(v[i] + 2.0 * (qw * uv[i] + uuv[i]) for i in range(3))


def _quat_mul(q1, q2):
    x1, y1, z1, w1 = q1
    x2, y2, z2, w2 = q2
    return (w1 * x2 + x1 * w2 + y1 * z2 - z1 * y2,
            w1 * y2 - x1 * z2 + y1 * w2 + z1 * x2,
            w1 * z2 + x1 * y2 - y1 * x2 + z1 * w2,
            w1 * w2 - x1 * x2 - y1 * y2 - z1 * z2)


NUM_PTS = 100000
PT_SEGS = NUM_PTS // CHUNK


def _sc_body(p2x_hbm, p2y_hbm, gidx_hbm, midx_hbm, cidx_hbm, pidx_hbm,
             depth_hbm, cam_hbm, rel_hbm, p3x_hbm, p3y_hbm, p3z_hbm,
             ref_hbm, out_hbm, tbl_hbm,
             gidx_v0, midx_v0, cidx_v0, pidx_v0, p2x_v0, p2y_v0, depth_v0,
             pts_v0, ref_v0, out0_v0, out1_v0, out2_v0,
             gidx_v1, midx_v1, cidx_v1, pidx_v1, p2x_v1, p2y_v1, depth_v1,
             pts_v1, ref_v1, out0_v1, out1_v1, out2_v1,
             cam_v, rel_v, sem_lin0, sem_gat0, sem_lin1, sem_gat1):
    cid = lax.axis_index("c")
    sid = lax.axis_index("s")
    wid = sid * 2 + cid
    pts_hbm = tbl_hbm.at[cid]
    pltpu.sync_copy(cam_hbm, cam_v)
    pltpu.sync_copy(rel_hbm, rel_v)

    for it2 in range(-(-PT_SEGS // 16)):
        seg = sid + it2 * 16

        @pl.when(seg < PT_SEGS)
        def _build():
            base = seg * CHUNK
            pltpu.sync_copy(p3x_hbm.at[pl.ds(base, CHUNK)], p2x_v0)
            pltpu.sync_copy(p3y_hbm.at[pl.ds(base, CHUNK)], p2y_v0)
            pltpu.sync_copy(p3z_hbm.at[pl.ds(base, CHUNK)], depth_v0)

            @plsc.parallel_loop(0, GROUPS, unroll=2)
            def _rows(j):
                rows = j * L + lax.iota(jnp.int32, L)
                for col, src in ((0, p2x_v0), (1, p2y_v0), (2, depth_v0)):
                    cols = jnp.full((L,), col, jnp.int32)
                    plsc.store_scatter(pts_v0, [rows, cols],
                                       src[pl.ds(j * L, L)])

            pltpu.sync_copy(pts_v0, pts_hbm.at[pl.ds(base, CHUNK)])

    plsc.subcore_barrier()

    sets = [
        dict(gidx=gidx_v0, midx=midx_v0, cidx=cidx_v0, pidx=pidx_v0,
             p2x=p2x_v0, p2y=p2y_v0, depth=depth_v0, pts=pts_v0, ref=ref_v0,
             out0=out0_v0, out1=out1_v0, out2=out2_v0,
             sem_lin=sem_lin0, sem_gat=sem_gat0),
        dict(gidx=gidx_v1, midx=midx_v1, cidx=cidx_v1, pidx=pidx_v1,
             p2x=p2x_v1, p2y=p2y_v1, depth=depth_v1, pts=pts_v1, ref=ref_v1,
             out0=out0_v1, out1=out1_v1, out2=out2_v1,
             sem_lin=sem_lin1, sem_gat=sem_gat1),
    ]

    def lin_pairs(c, s):
        base = c * CHUNK
        return [
            (gidx_hbm.at[pl.ds(base, CHUNK)], s["gidx"]),
            (midx_hbm.at[pl.ds(base, CHUNK)], s["midx"]),
            (cidx_hbm.at[pl.ds(base, CHUNK)], s["cidx"]),
            (pidx_hbm.at[pl.ds(base, CHUNK)], s["pidx"]),
            (p2x_hbm.at[pl.ds(base, CHUNK)], s["p2x"]),
            (p2y_hbm.at[pl.ds(base, CHUNK)], s["p2y"]),
            (depth_hbm.at[pl.ds(base, CHUNK)], s["depth"]),
        ]

    def start_stage1(c, s):
        for src, dst in lin_pairs(c, s):
            pltpu.async_copy(src, dst, s["sem_lin"])

    def wait_stage1(c, s):
        for src, dst in lin_pairs(c, s):
            pltpu.make_async_copy(src, dst, s["sem_lin"]).wait()

    def start_gathers(s):
        pltpu.async_copy(pts_hbm.at[s["pidx"]], s["pts"], s["sem_gat"])
        pltpu.async_copy(ref_hbm.at[s["gidx"]], s["ref"], s["sem_gat"])

    def wait_gathers(s):
        pltpu.make_async_copy(pts_hbm.at[s["pidx"]], s["pts"], s["sem_gat"]).wait()
        pltpu.make_async_copy(ref_hbm.at[s["gidx"]], s["ref"], s["sem_gat"]).wait()

    def compute(c, s):
        base = c * CHUNK

        @plsc.parallel_loop(0, GROUPS, unroll=2)
        def body(j):
            rows = j * L + lax.iota(jnp.int32, L)

            def lg(ref, idxv, col):
                cols = jnp.full((L,), col, jnp.int32)
                return plsc.load_gather(ref, [idxv, cols])

            rt = tuple(lg(s["ref"], rows, k) for k in range(3))
            rq = tuple(lg(s["ref"], rows, k) for k in range(3, 7))
            mi = s["midx"][pl.ds(j * L, L)]
            ci = s["cidx"][pl.ds(j * L, L)]
            lt = tuple(lg(rel_v, mi, k) for k in range(3))
            lq = tuple(lg(rel_v, mi, k) for k in range(3, 7))
            fx = lg(cam_v, ci, 0)
            fy = lg(cam_v, ci, 1)
            cx = lg(cam_v, ci, 2)
            cy = lg(cam_v, ci, 3)
            p = tuple(lg(s["pts"], rows, k) for k in range(3))

            rrt = _quat_rotate(lq, rt)
            t = tuple(lt[i] + rrt[i] for i in range(3))
            q = _quat_mul(lq, rq)
            pc = _quat_rotate(q, p)
            pcx = pc[0] + t[0]
            pcy = pc[1] + t[1]
            z = pc[2] + t[2]
            u = pcx / z * fx + cx
            v = pcy / z * fy + cy
            p2x = s["p2x"][pl.ds(j * L, L)]
            p2y = s["p2y"][pl.ds(j * L, L)]
            dref = s["depth"][pl.ds(j * L, L)]
            out0 = u - p2x
            out1 = v - p2y
            out2 = (1.0 / (z + EPS) - dref) * DEPTH_WEIGHT
            s["out0"][pl.ds(j * L, L)] = out0
            s["out1"][pl.ds(j * L, L)] = out1
            s["out2"][pl.ds(j * L, L)] = out2

        pltpu.sync_copy(s["out0"], out_hbm.at[0, pl.ds(base, CHUNK)])
        pltpu.sync_copy(s["out1"], out_hbm.at[1, pl.ds(base, CHUNK)])
        pltpu.sync_copy(s["out2"], out_hbm.at[2, pl.ds(base, CHUNK)])

    start_stage1(wid, sets[0])
    wait_stage1(wid, sets[0])
    start_gathers(sets[0])
    for it in range(ITERS):
        c = wid + it * NWORKERS
        cn = wid + (it + 1) * NWORKERS
        cur = sets[it % 2]
        nxt = sets[(it + 1) % 2]
        last = it == ITERS - 1

        def guarded(fn, cond):
            if cond is None:
                fn()
            else:
                pl.when(cond)(fn)

        if not last:
            next_cond = None if it + 1 < ITERS - 1 else (cn < NUM_CHUNKS)
            guarded(lambda: start_stage1(cn, nxt), next_cond)
            wait_gathers(cur)
            guarded(lambda: (wait_stage1(cn, nxt), start_gathers(nxt))[0],
                    next_cond)
            compute(c, cur)
        else:
            @pl.when(c < NUM_CHUNKS)
            def _tail():
                wait_gathers(cur)
                compute(c, cur)


_sc_call = functools.partial(
    pl.kernel,
    mesh=plsc.VectorSubcoreMesh(core_axis_name="c", subcore_axis_name="s"),
    out_type=(jax.ShapeDtypeStruct((3, N), jnp.float32),
              jax.ShapeDtypeStruct((2, NUM_PTS, 8), jnp.float32)),
    compiler_params=pltpu.CompilerParams(
        needs_layout_passes=False, use_tc_tiling_on_sc=False),
    scratch_types=(
        [
            pltpu.VMEM((CHUNK,), jnp.int32),
            pltpu.VMEM((CHUNK,), jnp.int32),
            pltpu.VMEM((CHUNK,), jnp.int32),
            pltpu.VMEM((CHUNK,), jnp.int32),
            pltpu.VMEM((CHUNK,), jnp.float32),
            pltpu.VMEM((CHUNK,), jnp.float32),
            pltpu.VMEM((CHUNK,), jnp.float32),
            pltpu.VMEM((CHUNK, 8), jnp.float32),
            pltpu.VMEM((CHUNK, 8), jnp.float32),
            pltpu.VMEM((CHUNK,), jnp.float32),
            pltpu.VMEM((CHUNK,), jnp.float32),
            pltpu.VMEM((CHUNK,), jnp.float32),
        ] * 2
        + [
            pltpu.VMEM((8, 4), jnp.float32),
            pltpu.VMEM((8, 8), jnp.float32),
            pltpu.SemaphoreType.DMA,
            pltpu.SemaphoreType.DMA,
            pltpu.SemaphoreType.DMA,
            pltpu.SemaphoreType.DMA,
        ]
    ),
)(_sc_body)


def kernel(points_2d, camera_indices, grouping_indices, point_indices,
           camera_pps, rel_poses, depths_ref, intrs, points_3d, ref_poses):
    gidx = grouping_indices[:, 0].astype(jnp.int32)
    midx = grouping_indices[:, 1].astype(jnp.int32)
    cidx = camera_indices.astype(jnp.int32)
    pidx = point_indices.astype(jnp.int32)
    p2x = points_2d[:, 0]
    p2y = points_2d[:, 1]
    p3x = points_3d[:, 0]
    p3y = points_3d[:, 1]
    p3z = points_3d[:, 2]
    ref8 = jnp.pad(ref_poses, ((0, 0), (0, 1)))
    rel8 = jnp.pad(rel_poses, ((0, 0), (0, 1)))
    cam4 = jnp.concatenate([intrs, camera_pps], axis=1)
    out, _ = _sc_call(p2x, p2y, gidx, midx, cidx, pidx, depths_ref,
                      cam4, rel8, p3x, p3y, p3z, ref8)
    return out.T


# speedup vs baseline: 1.5567x; 1.2744x over previous
import functools

import jax
import jax.numpy as jnp
from jax import lax
from jax.experimental import pallas as pl
from jax.experimental.pallas import tpu as pltpu
from jax.experimental.pallas import tpu_sc as plsc

N = 500000
CHUNK = 2000
NUM_CHUNKS = N // CHUNK
L = 16
GROUPS = CHUNK // L
NWORKERS = 32
ITERS = -(-NUM_CHUNKS // NWORKERS)
DEPTH_WEIGHT = 0.1
EPS = 1e-6


def _cross(a, b):
    return (a[1] * b[2] - a[2] * b[1],
            a[2] * b[0] - a[0] * b[2],
            a[0] * b[1] - a[1] * b[0])


def _quat_rotate(q, v):
    qv = (q[0], q[1], q[2])
    qw = q[3]
    uv = _cross(qv, v)
    uuv = _cross(qv, uv)
    return tuple(v[i] + 2.0 * (qw * uv[i] + uuv[i]) for i in range(3))


def _quat_mul(q1, q2):
    x1, y1, z1, w1 = q1
    x2, y2, z2, w2 = q2
    return (w1 * x2 + x1 * w2 + y1 * z2 - z1 * y2,
            w1 * y2 - x1 * z2 + y1 * w2 + z1 * x2,
            w1 * z2 + x1 * y2 - y1 * x2 + z1 * w2,
            w1 * w2 - x1 * x2 - y1 * y2 - z1 * z2)


NUM_PTS = 100000
PT_SEGS = NUM_PTS // CHUNK


def _sc_body(p2d_hbm, grp_hbm, cidx_hbm, pidx_hbm,
             depth_hbm, cam_hbm, rel_hbm, p3x_hbm, p3y_hbm, p3z_hbm,
             ref_hbm, out_hbm, tbl_hbm,
             gidx_v0, midx_v0, cidx_v0, pidx_v0, p2x_v0, p2y_v0, depth_v0,
             pts_v0, ref_v0, out0_v0, out1_v0, out2_v0,
             gidx_v1, midx_v1, cidx_v1, pidx_v1, p2x_v1, p2y_v1, depth_v1,
             pts_v1, ref_v1, out0_v1, out1_v1, out2_v1,
             cam_v, rel_v, sem_lin0, sem_gat0, sem_lin1, sem_gat1):
    cid = lax.axis_index("c")
    sid = lax.axis_index("s")
    wid = sid * 2 + cid
    pts_hbm = tbl_hbm.at[cid]
    pltpu.sync_copy(cam_hbm, cam_v)
    pltpu.sync_copy(rel_hbm, rel_v)

    for it2 in range(-(-PT_SEGS // 16)):
        seg = sid + it2 * 16

        @pl.when(seg < PT_SEGS)
        def _build():
            base = seg * CHUNK
            pltpu.sync_copy(p3x_hbm.at[pl.ds(base, CHUNK)], out0_v0)
            pltpu.sync_copy(p3y_hbm.at[pl.ds(base, CHUNK)], out1_v0)
            pltpu.sync_copy(p3z_hbm.at[pl.ds(base, CHUNK)], depth_v0)

            @plsc.parallel_loop(0, GROUPS, unroll=2)
            def _rows(j):
                rows = j * L + lax.iota(jnp.int32, L)
                for col, src in ((0, out0_v0), (1, out1_v0), (2, depth_v0)):
                    cols = jnp.full((L,), col, jnp.int32)
                    plsc.store_scatter(pts_v0, [rows, cols],
                                       src[pl.ds(j * L, L)])

            pltpu.sync_copy(pts_v0, pts_hbm.at[pl.ds(base, CHUNK)])

    plsc.subcore_barrier()

    sets = [
        dict(gidx=gidx_v0, midx=midx_v0, cidx=cidx_v0, pidx=pidx_v0,
             p2x=p2x_v0, p2y=p2y_v0, depth=depth_v0, pts=pts_v0, ref=ref_v0,
             out0=out0_v0, out1=out1_v0, out2=out2_v0,
             sem_lin=sem_lin0, sem_gat=sem_gat0),
        dict(gidx=gidx_v1, midx=midx_v1, cidx=cidx_v1, pidx=pidx_v1,
             p2x=p2x_v1, p2y=p2y_v1, depth=depth_v1, pts=pts_v1, ref=ref_v1,
             out0=out0_v1, out1=out1_v1, out2=out2_v1,
             sem_lin=sem_lin1, sem_gat=sem_gat1),
    ]

    def lin_pairs(c, s):
        base = c * CHUNK
        return [
            (grp_hbm.at[pl.ds(base, CHUNK)], s["gidx"]),
            (grp_hbm.at[pl.ds(N + base, CHUNK)], s["midx"]),
            (p2d_hbm.at[pl.ds(base, CHUNK)], s["p2x"]),
            (p2d_hbm.at[pl.ds(N + base, CHUNK)], s["p2y"]),
            (cidx_hbm.at[pl.ds(base, CHUNK)], s["cidx"]),
            (pidx_hbm.at[pl.ds(base, CHUNK)], s["pidx"]),
            (depth_hbm.at[pl.ds(base, CHUNK)], s["depth"]),
        ]

    def start_stage1(c, s):
        for src, dst in lin_pairs(c, s):
            pltpu.async_copy(src, dst, s["sem_lin"])

    def wait_stage1(c, s):
        for src, dst in lin_pairs(c, s):
            pltpu.make_async_copy(src, dst, s["sem_lin"]).wait()

    def start_gathers(s):
        pltpu.async_copy(pts_hbm.at[s["pidx"]], s["pts"], s["sem_gat"])
        pltpu.async_copy(ref_hbm.at[s["gidx"]], s["ref"], s["sem_gat"])

    def wait_gathers(s):
        pltpu.make_async_copy(pts_hbm.at[s["pidx"]], s["pts"], s["sem_gat"]).wait()
        pltpu.make_async_copy(ref_hbm.at[s["gidx"]], s["ref"], s["sem_gat"]).wait()

    def compute(c, s):
        base = c * CHUNK

        @plsc.parallel_loop(0, GROUPS, unroll=2)
        def body(j):
            rows = j * L + lax.iota(jnp.int32, L)

            def lg(ref, idxv, col):
                cols = jnp.full((L,), col, jnp.int32)
                return plsc.load_gather(ref, [idxv, cols])

            rt = tuple(lg(s["ref"], rows, k) for k in range(3))
            rq = tuple(lg(s["ref"], rows, k) for k in range(3, 7))
            mi = s["midx"][pl.ds(j * L, L)]
            ci = s["cidx"][pl.ds(j * L, L)]
            lt = tuple(lg(rel_v, mi, k) for k in range(3))
            lq = tuple(lg(rel_v, mi, k) for k in range(3, 7))
            fx = lg(cam_v, ci, 0)
            fy = lg(cam_v, ci, 1)
            cx = lg(cam_v, ci, 2)
            cy = lg(cam_v, ci, 3)
            p = tuple(lg(s["pts"], rows, k) for k in range(3))

            rrt = _quat_rotate(lq, rt)
            t = tuple(lt[i] + rrt[i] for i in range(3))
            q = _quat_mul(lq, rq)
            pc = _quat_rotate(q, p)
            pcx = pc[0] + t[0]
            pcy = pc[1] + t[1]
            z = pc[2] + t[2]
            u = pcx / z * fx + cx
            v = pcy / z * fy + cy
            p2x = s["p2x"][pl.ds(j * L, L)]
            p2y = s["p2y"][pl.ds(j * L, L)]
            dref = s["depth"][pl.ds(j * L, L)]
            out0 = u - p2x
            out1 = v - p2y
            out2 = (1.0 / (z + EPS) - dref) * DEPTH_WEIGHT
            s["out0"][pl.ds(j * L, L)] = out0
            s["out1"][pl.ds(j * L, L)] = out1
            s["out2"][pl.ds(j * L, L)] = out2

        pltpu.sync_copy(s["out0"], out_hbm.at[0, pl.ds(base, CHUNK)])
        pltpu.sync_copy(s["out1"], out_hbm.at[1, pl.ds(base, CHUNK)])
        pltpu.sync_copy(s["out2"], out_hbm.at[2, pl.ds(base, CHUNK)])

    start_stage1(wid, sets[0])
    wait_stage1(wid, sets[0])
    start_gathers(sets[0])
    for it in range(ITERS):
        c = wid + it * NWORKERS
        cn = wid + (it + 1) * NWORKERS
        cur = sets[it % 2]
        nxt = sets[(it + 1) % 2]
        last = it == ITERS - 1

        def guarded(fn, cond):
            if cond is None:
                fn()
            else:
                pl.when(cond)(fn)

        if not last:
            next_cond = None if it + 1 < ITERS - 1 else (cn < NUM_CHUNKS)
            guarded(lambda: start_stage1(cn, nxt), next_cond)
            wait_gathers(cur)
            guarded(lambda: (wait_stage1(cn, nxt), start_gathers(nxt))[0],
                    next_cond)
            compute(c, cur)
        else:
            @pl.when(c < NUM_CHUNKS)
            def _tail():
                wait_gathers(cur)
                compute(c, cur)


_sc_call = functools.partial(
    pl.kernel,
    mesh=plsc.VectorSubcoreMesh(core_axis_name="c", subcore_axis_name="s"),
    out_type=(jax.ShapeDtypeStruct((3, N), jnp.float32),
              jax.ShapeDtypeStruct((2, NUM_PTS, 8), jnp.float32)),
    compiler_params=pltpu.CompilerParams(
        needs_layout_passes=False, use_tc_tiling_on_sc=False),
    scratch_types=(
        [
            pltpu.VMEM((CHUNK,), jnp.int32),
            pltpu.VMEM((CHUNK,), jnp.int32),
            pltpu.VMEM((CHUNK,), jnp.int32),
            pltpu.VMEM((CHUNK,), jnp.int32),
            pltpu.VMEM((CHUNK,), jnp.float32),
            pltpu.VMEM((CHUNK,), jnp.float32),
            pltpu.VMEM((CHUNK,), jnp.float32),
            pltpu.VMEM((CHUNK, 8), jnp.float32),
            pltpu.VMEM((CHUNK, 8), jnp.float32),
            pltpu.VMEM((CHUNK,), jnp.float32),
            pltpu.VMEM((CHUNK,), jnp.float32),
            pltpu.VMEM((CHUNK,), jnp.float32),
        ] * 2
        + [
            pltpu.VMEM((8, 4), jnp.float32),
            pltpu.VMEM((8, 8), jnp.float32),
            pltpu.SemaphoreType.DMA,
            pltpu.SemaphoreType.DMA,
            pltpu.SemaphoreType.DMA,
            pltpu.SemaphoreType.DMA,
        ]
    ),
)(_sc_body)


def kernel(points_2d, camera_indices, grouping_indices, point_indices,
           camera_pps, rel_poses, depths_ref, intrs, points_3d, ref_poses):
    grp = grouping_indices.astype(jnp.int32).T.reshape(-1)
    cidx = camera_indices.astype(jnp.int32)
    pidx = point_indices.astype(jnp.int32)
    p2d = points_2d.T.reshape(-1)
    p3x = points_3d[:, 0]
    p3y = points_3d[:, 1]
    p3z = points_3d[:, 2]
    ref8 = jnp.pad(ref_poses, ((0, 0), (0, 1)))
    rel8 = jnp.pad(rel_poses, ((0, 0), (0, 1)))
    cam4 = jnp.concatenate([intrs, camera_pps], axis=1)
    out, _ = _sc_call(p2d, grp, cidx, pidx, depths_ref,
                      cam4, rel8, p3x, p3y, p3z, ref8)
    return out.T
